# baseline (speedup 1.0000x reference)
import functools

import jax
import jax.numpy as jnp
from jax import lax
from jax.experimental import pallas as pl
from jax.experimental.pallas import tpu as pltpu
from jax.experimental.pallas import tpu_sc as plsc

N = 10000
E = 320000
D = 128

NC = 2
NS = 16
NW = NC * NS
CHUNK = 128
NCH = 80
EPW = NCH * CHUNK
E_PAD = EPW * NW
RPT = -(-(N + 1) // (NS * 8)) * 8
ACC_ROWS = RPT * NS
CW = 16


def _sc_body(with_counts, blk, *refs):
    if with_counts:
        (h, srcg, dstg, zacc, zcnt, agg_out, cnt_out,
         src_v, dst_v, rows_a, rows_b, cnt_priv, acc_sh, sem_a, sem_b) = refs
    else:
        (h, srcg, dstg, zacc, agg_out,
         src_v, dst_v, rows_a, rows_b, acc_sh, sem_a, sem_b) = refs
    c = lax.axis_index("c")
    s = lax.axis_index("s")
    gwid = c * NS + s

    pltpu.sync_copy(zacc, acc_sh.at[pl.ds(s * RPT, RPT)])
    if with_counts:
        pltpu.sync_copy(zcnt, cnt_priv)
    plsc.subcore_barrier()

    ones16 = jnp.ones((16,), jnp.float32)

    def fire(j, buf, sem):
        pltpu.async_copy(h.at[src_v.at[j]], buf, sem)

    def drain(buf, sem):
        pltpu.make_async_copy(h.at[src_v.at[0]], buf, sem).wait()

    def scat(j, buf):
        pltpu.sync_copy(buf, acc_sh.at[dst_v.at[j]], add=True)
        if with_counts:
            for k in range(CHUNK // 16):
                idx = dst_v[j, pl.ds(k * 16, 16)]
                plsc.addupdate_scatter(cnt_priv, [idx], ones16)

    def block(nb, carry):
        pltpu.sync_copy(srcg.at[gwid, pl.ds(nb * blk, blk)], src_v)
        pltpu.sync_copy(dstg.at[gwid, pl.ds(nb * blk, blk)], dst_v)
        fire(0, rows_a, sem_a)

        def step(i, carry2):
            j0 = 2 * i
            drain(rows_a, sem_a)
            fire(j0 + 1, rows_b, sem_b)
            scat(j0, rows_a)
            drain(rows_b, sem_b)
            fire(j0 + 2, rows_a, sem_a)
            scat(j0 + 1, rows_b)
            return carry2

        lax.fori_loop(0, blk // 2 - 1, step, 0)
        drain(rows_a, sem_a)
        fire(blk - 1, rows_b, sem_b)
        scat(blk - 2, rows_a)
        drain(rows_b, sem_b)
        scat(blk - 1, rows_b)
        return carry

    lax.fori_loop(0, NCH // blk, block, 0)
    plsc.subcore_barrier()

    base = s * RPT
    last = N - (NS - 1) * RPT

    @pl.when(s < NS - 1)
    def _():
        pltpu.sync_copy(acc_sh.at[pl.ds(base, RPT)],
                        agg_out.at[c, pl.ds(base, RPT)])

    @pl.when(s == NS - 1)
    def _():
        pltpu.sync_copy(acc_sh.at[pl.ds(base, last)],
                        agg_out.at[c, pl.ds(base, last)])

    if with_counts:
        pltpu.sync_copy(cnt_priv, cnt_out.at[c, s])


BLK_CNT = 8
BLK = 16


@functools.lru_cache(maxsize=None)
def _sc_kernels():
    mesh = plsc.VectorSubcoreMesh(core_axis_name="c", subcore_axis_name="s",
                                  num_cores=NC, num_subcores=NS)

    def scratch(blk):
        return [
            pltpu.VMEM((blk, CHUNK), jnp.int32),
            pltpu.VMEM((blk, CHUNK), jnp.int32),
            pltpu.VMEM((CHUNK, D), jnp.float32),
            pltpu.VMEM((CHUNK, D), jnp.float32),
        ]

    params = pltpu.CompilerParams(needs_layout_passes=False)
    agg_cnt = pl.kernel(
        functools.partial(_sc_body, True, BLK_CNT),
        out_type=(jax.ShapeDtypeStruct((NC, N, D), jnp.float32),
                  jax.ShapeDtypeStruct((NC, NS, ACC_ROWS), jnp.float32)),
        mesh=mesh,
        compiler_params=params,
        scratch_types=scratch(BLK_CNT) + [
            pltpu.VMEM((ACC_ROWS,), jnp.float32),
            pltpu.VMEM_SHARED((ACC_ROWS, D), jnp.float32),
            pltpu.SemaphoreType.DMA,
            pltpu.SemaphoreType.DMA,
        ],
    )
    agg = pl.kernel(
        functools.partial(_sc_body, False, BLK),
        out_type=jax.ShapeDtypeStruct((NC, N, D), jnp.float32),
        mesh=mesh,
        compiler_params=params,
        scratch_types=scratch(BLK) + [
            pltpu.VMEM_SHARED((ACC_ROWS, D), jnp.float32),
            pltpu.SemaphoreType.DMA,
            pltpu.SemaphoreType.DMA,
        ],
    )
    return agg_cnt, agg


RB = 1000


def _tc_body(relu, p_ref, cnt_ref, h_ref, wlt_ref, wrt_ref, bl_ref, out_ref):
    cnt = jnp.sum(cnt_ref[...], axis=1, keepdims=True)
    rcp = 1.0 / jnp.maximum(cnt, 1.0)
    mean = (p_ref[0] + p_ref[1]) * rcp
    out = (jnp.dot(mean, wlt_ref[...], preferred_element_type=jnp.float32)
           + jnp.dot(h_ref[...], wrt_ref[...],
                     preferred_element_type=jnp.float32)
           + bl_ref[...])
    if relu:
        out = jnp.maximum(out, 0.0)
    out_ref[...] = out


def _tc_layer(p, cntp, h, wlt, wrt, bl, relu):
    grid = (N // RB,)
    return pl.pallas_call(
        functools.partial(_tc_body, relu),
        grid=grid,
        in_specs=[
            pl.BlockSpec((NC, RB, D), lambda i: (0, i, 0)),
            pl.BlockSpec((RB, NW), lambda i: (i, 0)),
            pl.BlockSpec((RB, D), lambda i: (i, 0)),
            pl.BlockSpec((D, D), lambda i: (0, 0)),
            pl.BlockSpec((D, D), lambda i: (0, 0)),
            pl.BlockSpec((1, D), lambda i: (0, 0)),
        ],
        out_specs=pl.BlockSpec((RB, D), lambda i: (i, 0)),
        out_shape=jax.ShapeDtypeStruct((N, D), jnp.float32),
    )(p, cntp, h, wlt, wrt, bl)


def kernel(x, edge_index, Wl0, bl0, Wr0, Wl1, bl1, Wr1, Wl2, bl2, Wr2):
    src = edge_index[0]
    dst = edge_index[1]
    pad = E_PAD - E
    src_p = jnp.concatenate([src, jnp.zeros((pad,), jnp.int32)])
    dst_p = jnp.concatenate([dst, jnp.full((pad,), N, jnp.int32)])
    srcg = src_p.reshape(NW, NCH, CHUNK)
    dstg = dst_p.reshape(NW, NCH, CHUNK)
    zacc = jnp.zeros((RPT, D), jnp.float32)
    zcnt = jnp.zeros((ACC_ROWS,), jnp.float32)

    sc_agg_cnt, sc_agg = _sc_kernels()
    a0, cntp = sc_agg_cnt(x, srcg, dstg, zacc, zcnt)
    cntp = cntp.reshape(NW, ACC_ROWS).T
    h1 = _tc_layer(a0, cntp, x, Wl0.T, Wr0.T, bl0.reshape(1, D), relu=True)
    a1 = sc_agg(h1, srcg, dstg, zacc)
    h2 = _tc_layer(a1, cntp, h1, Wl1.T, Wr1.T, bl1.reshape(1, D), relu=True)
    a2 = sc_agg(h2, srcg, dstg, zacc)
    h3 = _tc_layer(a2, cntp, h2, Wl2.T, Wr2.T, bl2.reshape(1, D), relu=False)
    return h3


# speedup vs baseline: 1.0215x; 1.0215x over previous
import functools

import jax
import jax.numpy as jnp
from jax import lax
from jax.experimental import pallas as pl
from jax.experimental.pallas import tpu as pltpu
from jax.experimental.pallas import tpu_sc as plsc

N = 10000
E = 320000
D = 128

NC = 2
NS = 16
NW = NC * NS
CHUNK = 128
NCH = 80
EPW = NCH * CHUNK
E_PAD = EPW * NW
RPT = -(-(N + 1) // (NS * 8)) * 8
ACC_ROWS = RPT * NS
CW = 16


def _sc_body(with_counts, blk, *refs):
    if with_counts:
        (h, srcg, dstg, zacc, zcnt, agg_out, cnt_out,
         src_v, dst_v, rows_a, rows_b, cnt_priv, acc_sh,
         sem_a, sem_b, sem_sa, sem_sb) = refs
    else:
        (h, srcg, dstg, zacc, agg_out,
         src_v, dst_v, rows_a, rows_b, acc_sh,
         sem_a, sem_b, sem_sa, sem_sb) = refs
    c = lax.axis_index("c")
    s = lax.axis_index("s")
    gwid = c * NS + s

    pltpu.sync_copy(zacc, acc_sh.at[pl.ds(s * RPT, RPT)])
    if with_counts:
        pltpu.sync_copy(zcnt, cnt_priv)
    plsc.subcore_barrier()

    ones16 = jnp.ones((16,), jnp.float32)

    def fire(j, buf, sem):
        pltpu.async_copy(h.at[src_v.at[j]], buf, sem)

    def drain(buf, sem):
        pltpu.make_async_copy(h.at[src_v.at[0]], buf, sem).wait()

    def fire_scat(j, buf, sem):
        pltpu.async_copy(buf, acc_sh.at[dst_v.at[j]], sem, add=True)
        if with_counts:
            for k in range(CHUNK // 16):
                idx = dst_v[j, pl.ds(k * 16, 16)]
                plsc.addupdate_scatter(cnt_priv, [idx], ones16)

    def drain_scat(buf, sem):
        pltpu.make_async_copy(buf, acc_sh.at[dst_v.at[0]], sem).wait()

    def block(nb, carry):
        pltpu.sync_copy(srcg.at[gwid, pl.ds(nb * blk, blk)], src_v)
        pltpu.sync_copy(dstg.at[gwid, pl.ds(nb * blk, blk)], dst_v)
        fire(0, rows_a, sem_a)
        fire(1, rows_b, sem_b)

        def step(i, carry2):
            j0 = 2 * i
            drain(rows_a, sem_a)
            fire_scat(j0, rows_a, sem_sa)
            drain(rows_b, sem_b)
            fire_scat(j0 + 1, rows_b, sem_sb)
            drain_scat(rows_a, sem_sa)
            fire(j0 + 2, rows_a, sem_a)
            drain_scat(rows_b, sem_sb)
            fire(j0 + 3, rows_b, sem_b)
            return carry2

        lax.fori_loop(0, blk // 2 - 1, step, 0)
        drain(rows_a, sem_a)
        fire_scat(blk - 2, rows_a, sem_sa)
        drain(rows_b, sem_b)
        fire_scat(blk - 1, rows_b, sem_sb)
        drain_scat(rows_a, sem_sa)
        drain_scat(rows_b, sem_sb)
        return carry

    lax.fori_loop(0, NCH // blk, block, 0)
    plsc.subcore_barrier()

    base = s * RPT
    last = N - (NS - 1) * RPT

    @pl.when(s < NS - 1)
    def _():
        pltpu.sync_copy(acc_sh.at[pl.ds(base, RPT)],
                        agg_out.at[c, pl.ds(base, RPT)])

    @pl.when(s == NS - 1)
    def _():
        pltpu.sync_copy(acc_sh.at[pl.ds(base, last)],
                        agg_out.at[c, pl.ds(base, last)])

    if with_counts:
        pltpu.sync_copy(cnt_priv, cnt_out.at[c, s])


BLK_CNT = 8
BLK = 16


@functools.lru_cache(maxsize=None)
def _sc_kernels():
    mesh = plsc.VectorSubcoreMesh(core_axis_name="c", subcore_axis_name="s",
                                  num_cores=NC, num_subcores=NS)

    def scratch(blk):
        return [
            pltpu.VMEM((blk, CHUNK), jnp.int32),
            pltpu.VMEM((blk, CHUNK), jnp.int32),
            pltpu.VMEM((CHUNK, D), jnp.float32),
            pltpu.VMEM((CHUNK, D), jnp.float32),
        ]

    params = pltpu.CompilerParams(needs_layout_passes=False)
    agg_cnt = pl.kernel(
        functools.partial(_sc_body, True, BLK_CNT),
        out_type=(jax.ShapeDtypeStruct((NC, N, D), jnp.float32),
                  jax.ShapeDtypeStruct((NC, NS, ACC_ROWS), jnp.float32)),
        mesh=mesh,
        compiler_params=params,
        scratch_types=scratch(BLK_CNT) + [
            pltpu.VMEM((ACC_ROWS,), jnp.float32),
            pltpu.VMEM_SHARED((ACC_ROWS, D), jnp.float32),
            pltpu.SemaphoreType.DMA,
            pltpu.SemaphoreType.DMA,
            pltpu.SemaphoreType.DMA,
            pltpu.SemaphoreType.DMA,
        ],
    )
    agg = pl.kernel(
        functools.partial(_sc_body, False, BLK),
        out_type=jax.ShapeDtypeStruct((NC, N, D), jnp.float32),
        mesh=mesh,
        compiler_params=params,
        scratch_types=scratch(BLK) + [
            pltpu.VMEM_SHARED((ACC_ROWS, D), jnp.float32),
            pltpu.SemaphoreType.DMA,
            pltpu.SemaphoreType.DMA,
            pltpu.SemaphoreType.DMA,
            pltpu.SemaphoreType.DMA,
        ],
    )
    return agg_cnt, agg


RB = 1000


def _tc_body(relu, p_ref, cnt_ref, h_ref, wlt_ref, wrt_ref, bl_ref, out_ref):
    cnt = jnp.sum(cnt_ref[...], axis=1, keepdims=True)
    rcp = 1.0 / jnp.maximum(cnt, 1.0)
    mean = (p_ref[0] + p_ref[1]) * rcp
    out = (jnp.dot(mean, wlt_ref[...], preferred_element_type=jnp.float32)
           + jnp.dot(h_ref[...], wrt_ref[...],
                     preferred_element_type=jnp.float32)
           + bl_ref[...])
    if relu:
        out = jnp.maximum(out, 0.0)
    out_ref[...] = out


def _tc_layer(p, cntp, h, wlt, wrt, bl, relu):
    grid = (N // RB,)
    return pl.pallas_call(
        functools.partial(_tc_body, relu),
        grid=grid,
        in_specs=[
            pl.BlockSpec((NC, RB, D), lambda i: (0, i, 0)),
            pl.BlockSpec((RB, NW), lambda i: (i, 0)),
            pl.BlockSpec((RB, D), lambda i: (i, 0)),
            pl.BlockSpec((D, D), lambda i: (0, 0)),
            pl.BlockSpec((D, D), lambda i: (0, 0)),
            pl.BlockSpec((1, D), lambda i: (0, 0)),
        ],
        out_specs=pl.BlockSpec((RB, D), lambda i: (i, 0)),
        out_shape=jax.ShapeDtypeStruct((N, D), jnp.float32),
    )(p, cntp, h, wlt, wrt, bl)


def kernel(x, edge_index, Wl0, bl0, Wr0, Wl1, bl1, Wr1, Wl2, bl2, Wr2):
    src = edge_index[0]
    dst = edge_index[1]
    pad = E_PAD - E
    src_p = jnp.concatenate([src, jnp.zeros((pad,), jnp.int32)])
    dst_p = jnp.concatenate([dst, jnp.full((pad,), N, jnp.int32)])
    srcg = src_p.reshape(NW, NCH, CHUNK)
    dstg = dst_p.reshape(NW, NCH, CHUNK)
    zacc = jnp.zeros((RPT, D), jnp.float32)
    zcnt = jnp.zeros((ACC_ROWS,), jnp.float32)

    sc_agg_cnt, sc_agg = _sc_kernels()
    a0, cntp = sc_agg_cnt(x, srcg, dstg, zacc, zcnt)
    cntp = cntp.reshape(NW, ACC_ROWS).T
    h1 = _tc_layer(a0, cntp, x, Wl0.T, Wr0.T, bl0.reshape(1, D), relu=True)
    a1 = sc_agg(h1, srcg, dstg, zacc)
    h2 = _tc_layer(a1, cntp, h1, Wl1.T, Wr1.T, bl1.reshape(1, D), relu=True)
    a2 = sc_agg(h2, srcg, dstg, zacc)
    h3 = _tc_layer(a2, cntp, h2, Wl2.T, Wr2.T, bl2.reshape(1, D), relu=False)
    return h3
